# Initial kernel scaffold; baseline (speedup 1.0000x reference)
import jax
import jax.numpy as jnp
from jax.experimental import pallas as pl


def kernel(X_v, edge_index, attW0, attq0, W0, b0, attW1, attq1, W1, b1, Wout, bout):
    raise NotImplementedError("write your pallas kernel here")


# speedup vs baseline: 44.3460x; 44.3460x over previous
import jax
import jax.numpy as jnp
from jax import lax
from jax.experimental import pallas as pl
from jax.experimental.pallas import tpu as pltpu
from jax.experimental.pallas import tpu_sc as plsc

N = 10000
E = 320000
NPAD = 10240
DST_NODE = 14
L = 16
NT = 16
C = E // NT
CHUNK = 2000
NCHUNK = C // CHUNK
CAP = C + 2 * L
SENT = NPAD - 8
NSEG = 4
SEGR = NPAD // NSEG
SSENT = SEGR - 8
NEG = -3.0e38

f32 = jnp.float32
i32 = jnp.int32


_DNUMS = lax.GatherDimensionNumbers(
    offset_dims=(), collapsed_slice_dims=(0,), start_index_map=(0,))


def _take16(c, idx):
    return lax.gather(c, idx[:, None], _DNUMS, slice_sizes=(1,),
                      mode=lax.GatherScatterMode.PROMISE_IN_BOUNDS)


def _prefix16(m):
    c = jnp.where(m, 1, 0).astype(i32)
    idx = lax.iota(i32, L)
    for sh in (1, 2, 4, 8):
        g = _take16(c, jnp.maximum(idx - sh, 0))
        c = c + jnp.where(idx >= sh, g, 0)
    return c


def _tc_scores_body(x_ref, w_ref, q_ref, o_ref):
    h = jnp.maximum(
        jnp.dot(x_ref[...], w_ref[...], preferred_element_type=f32), 0.0)
    s = jnp.dot(h, q_ref[...], preferred_element_type=f32)
    o_ref[...] = s.reshape(8, 128)


def _tc_scores(xp, attW0, attq0):
    grid = NPAD // 1024
    return pl.pallas_call(
        _tc_scores_body,
        grid=(grid,),
        in_specs=[
            pl.BlockSpec((1024, 128), lambda i: (i, 0)),
            pl.BlockSpec((128, 64), lambda i: (0, 0)),
            pl.BlockSpec((64, 1), lambda i: (0, 0)),
        ],
        out_specs=pl.BlockSpec((8, 128), lambda i: (i, 0)),
        out_shape=jax.ShapeDtypeStruct((NPAD // 128, 128), f32),
    )(xp, attW0, attq0.reshape(64, 1))


def _tc_hidden_body(x_ref, hn_ref, w0_ref, b0_ref, aw_ref, aq_ref, o_ref):
    a = jnp.dot(x_ref[...], w0_ref[0:128, :], preferred_element_type=f32)
    a = a + jnp.dot(hn_ref[...], w0_ref[128:256, :], preferred_element_type=f32)
    h1 = jnp.maximum(a + b0_ref[...], 0.0)
    t = jnp.maximum(jnp.dot(h1, aw_ref[...], preferred_element_type=f32), 0.0)
    t1 = jnp.dot(t, aq_ref[...], preferred_element_type=f32)
    o_ref[...] = jnp.concatenate(
        [h1, t1, jnp.zeros((512, 63), f32)], axis=1)


def _tc_hidden(xp, hn, W0, b0, attW1, attq1):
    grid = NPAD // 512
    return pl.pallas_call(
        _tc_hidden_body,
        grid=(grid,),
        in_specs=[
            pl.BlockSpec((512, 128), lambda i: (i, 0)),
            pl.BlockSpec((512, 128), lambda i: (i, 0)),
            pl.BlockSpec((256, 64), lambda i: (0, 0)),
            pl.BlockSpec((1, 64), lambda i: (0, 0)),
            pl.BlockSpec((64, 32), lambda i: (0, 0)),
            pl.BlockSpec((32, 1), lambda i: (0, 0)),
        ],
        out_specs=pl.BlockSpec((512, 128), lambda i: (i, 0)),
        out_shape=jax.ShapeDtypeStruct((NPAD, 128), f32),
    )(xp, hn, W0, b0.reshape(1, 64), attW1, attq1.reshape(32, 1))


def _sc_layer1_body(src_hbm, dst_hbm, t0_hbm, x_hbm,
                    hn_hbm, sel_hbm, cnt_hbm,
                    ebs, ebd, l14, bmp, t0v, ssrc, sdst, dnv,
                    xrows, zbuf, exb, cb, mxb,
                    sp_hn, sp_dn, sp_c, sp_m, sem):
    sid = lax.axis_index("s")
    i16 = lax.iota(i32, L)
    zf = jnp.zeros((L,), f32)
    zi = jnp.zeros((L,), i32)
    ones = jnp.ones((L,), i32)

    def _z(i, _):
        bmp[pl.ds(i * L, L)] = zi
        dnv[pl.ds(i * L, L)] = zf
        return 0
    lax.fori_loop(0, NPAD // L, _z, 0)

    for r in range(L):
        for jj in range(8):
            zbuf[r, pl.ds(jj * L, L)] = zf

    rows_per_tile = NPAD // NT
    pltpu.sync_copy(dnv.at[pl.ds(0, rows_per_tile)],
                    sp_dn.at[pl.ds(sid * rows_per_tile, rows_per_tile)])

    pltpu.sync_copy(t0_hbm, t0v)

    def _c1(c, k14):
        base = sid * C + c * CHUNK
        pltpu.sync_copy(src_hbm.at[pl.ds(base, CHUNK)], ebs)
        pltpu.sync_copy(dst_hbm.at[pl.ds(base, CHUNK)], ebd)
        def _v(i, k):
            d = ebd[pl.ds(i * L, L)]
            s = ebs[pl.ds(i * L, L)]
            m = d == DST_NODE
            cum = _prefix16(m)
            plsc.store_scatter(l14, [k + cum - 1], s, mask=m)
            return k + cum[15]
        return lax.fori_loop(0, CHUNK // L, _v, k14)
    k14 = lax.fori_loop(0, NCHUNK, _c1, jnp.asarray(0, i32))

    pltpu.sync_copy(l14, sel_hbm.at[pl.ds(sid * CAP, CAP)])
    cb[...] = jnp.full((L,), 0, i32) + k14
    pltpu.sync_copy(cb, cnt_hbm.at[pl.ds(sid * L, L)])
    pltpu.sync_copy(cb, sp_c.at[pl.ds(sid * L, L)])
    plsc.subcore_barrier()

    plsc.store_scatter(bmp, [jnp.full((L,), DST_NODE, i32)], ones)
    def _t(j, _):
        pltpu.sync_copy(sp_c.at[pl.ds(j * L, L)], cb)
        kj = cb[pl.ds(0, L)][0]
        def _m(q, _):
            pltpu.sync_copy(sel_hbm.at[pl.ds(j * CAP + q * CHUNK, CHUNK)], ebd)
            rem = jnp.minimum(kj - q * CHUNK, CHUNK)
            def _v(i, _):
                idx = ebd[pl.ds(i * L, L)]
                m = (i * L + i16) < rem
                idx = jnp.where(m, idx, 0)
                plsc.store_scatter(bmp, [idx], ones, mask=m)
                return 0
            lax.fori_loop(0, (rem + L - 1) // L, _v, 0)
            return 0
        lax.fori_loop(0, (kj + CHUNK - 1) // CHUNK, _m, 0)
        return 0
    lax.fori_loop(0, NT, _t, 0)

    def _c2(c, carry):
        base = sid * C + c * CHUNK
        pltpu.sync_copy(src_hbm.at[pl.ds(base, CHUNK)], ebs)
        pltpu.sync_copy(dst_hbm.at[pl.ds(base, CHUNK)], ebd)
        def _v(i, carry):
            et, mx = carry
            d = ebd[pl.ds(i * L, L)]
            s = ebs[pl.ds(i * L, L)]
            hit = plsc.load_gather(bmp, [d]) > 0
            cum = _prefix16(hit)
            pos = et + cum - 1
            plsc.store_scatter(ssrc, [pos], s, mask=hit)
            plsc.store_scatter(sdst, [pos], d, mask=hit)
            sv = plsc.load_gather(t0v, [s])
            mx = jnp.maximum(mx, jnp.where(hit, sv, NEG))
            return (et + cum[15], mx)
        return lax.fori_loop(0, CHUNK // L, _v, carry)
    et, mxv = lax.fori_loop(0, NCHUNK, _c2,
                            (jnp.asarray(0, i32), jnp.full((L,), NEG, f32)))

    mxb[...] = mxv
    pltpu.sync_copy(mxb, sp_m.at[pl.ds(sid * L, L)])
    plsc.subcore_barrier()
    def _r(j, g):
        pltpu.sync_copy(sp_m.at[pl.ds(j * L, L)], mxb)
        return jnp.maximum(g, mxb[...])
    gmax = jnp.max(lax.fori_loop(0, NT, _r, jnp.full((L,), NEG, f32)))

    nv = (et + L - 1) // L
    def _d(i, _):
        m = (i * L + i16) < et
        s = jnp.where(m, ssrc[pl.ds(i * L, L)], 0)
        d = jnp.where(m, sdst[pl.ds(i * L, L)], SENT)
        sv = plsc.load_gather(t0v, [s])
        exb[...] = jnp.where(m, jnp.exp(sv - gmax), 0.0)
        pltpu.sync_copy(exb, sp_dn.at[d], add=True)
        return 0
    lax.fori_loop(0, nv, _d, 0)
    plsc.subcore_barrier()
    pltpu.sync_copy(sp_dn, dnv)

    seg_per_tile = SEGR // NT
    for seg in range(NSEG):
        lo = seg * SEGR
        def _zs(q, _):
            pltpu.sync_copy(
                zbuf, sp_hn.at[pl.ds(sid * seg_per_tile + q * L, L)])
            return 0
        lax.fori_loop(0, seg_per_tile // L, _zs, 0)
        plsc.subcore_barrier()

        def _w(i, _):
            m = (i * L + i16) < et
            s = jnp.where(m, ssrc[pl.ds(i * L, L)], 0)
            d = jnp.where(m, sdst[pl.ds(i * L, L)], SENT)
            ms = m & (d >= lo) & (d < lo + SEGR)
            dloc = jnp.where(ms, d - lo, SSENT)
            sv = plsc.load_gather(t0v, [s])
            ex = jnp.where(ms, jnp.exp(sv - gmax), 0.0)
            dn = plsc.load_gather(dnv, [jnp.where(m, d, SENT)])
            w = ex / jnp.maximum(dn, 1e-16)
            pltpu.async_copy(x_hbm.at[s], xrows, sem).wait()
            for r in range(L):
                wr = w[r]
                for jj in range(8):
                    xrows[r, pl.ds(jj * L, L)] = (
                        xrows[r, pl.ds(jj * L, L)] * wr)
            pltpu.sync_copy(xrows, sp_hn.at[dloc], add=True)
            return 0
        lax.fori_loop(0, nv, _w, 0)
        plsc.subcore_barrier()

        pltpu.sync_copy(
            sp_hn.at[pl.ds(sid * seg_per_tile, seg_per_tile)],
            hn_hbm.at[pl.ds(lo + sid * seg_per_tile, seg_per_tile)])
        plsc.subcore_barrier()


def _sc_layer1(src, dst, t0, x):
    mesh = plsc.VectorSubcoreMesh(
        core_axis_name="c", subcore_axis_name="s", num_cores=1)
    fn = pl.kernel(
        _sc_layer1_body,
        out_type=(
            jax.ShapeDtypeStruct((NPAD, 128), f32),
            jax.ShapeDtypeStruct((NT * CAP,), i32),
            jax.ShapeDtypeStruct((NT * L,), i32),
        ),
        mesh=mesh,
        scratch_types=[
            pltpu.VMEM((CHUNK,), i32),
            pltpu.VMEM((CHUNK,), i32),
            pltpu.VMEM((CAP,), i32),
            pltpu.VMEM((NPAD,), i32),
            pltpu.VMEM((NPAD,), f32),
            pltpu.VMEM((CAP,), i32),
            pltpu.VMEM((CAP,), i32),
            pltpu.VMEM((NPAD,), f32),
            pltpu.VMEM((L, 128), f32),
            pltpu.VMEM((L, 128), f32),
            pltpu.VMEM((L,), f32),
            pltpu.VMEM((L,), i32),
            pltpu.VMEM((L,), f32),
            pltpu.VMEM_SHARED((SEGR, 128), f32),
            pltpu.VMEM_SHARED((NPAD,), f32),
            pltpu.VMEM_SHARED((NT * L,), i32),
            pltpu.VMEM_SHARED((NT * L,), f32),
            pltpu.SemaphoreType.DMA,
        ],
        compiler_params=pltpu.CompilerParams(needs_layout_passes=False),
    )
    return fn(src, dst, t0, x)


def _sc_layer2_body(sel_hbm, cnt_hbm, h1_hbm, w1_hbm, b1_hbm, wo_hbm, bo_hbm,
                    out_hbm,
                    mylist, xrows, cb, mxb, hac, w1v, b1v, wov, bov,
                    xcat, h2b, outb,
                    sp_m, sp_h, sp_d, sem):
    sid = lax.axis_index("s")
    i16 = lax.iota(i32, L)
    col64 = jnp.full((L,), 64, i32)
    zf = jnp.zeros((L,), f32)

    pltpu.sync_copy(cnt_hbm.at[pl.ds(sid * L, L)], cb)
    k = cb[pl.ds(0, L)][0]
    pltpu.sync_copy(sel_hbm.at[pl.ds(sid * CAP, CAP)], mylist)
    nv = (k + L - 1) // L

    def _a(i, mx):
        m = (i * L + i16) < k
        s = jnp.where(m, mylist[pl.ds(i * L, L)], 0)
        pltpu.async_copy(h1_hbm.at[s], xrows, sem).wait()
        t1v = plsc.load_gather(xrows, [i16, col64])
        return jnp.maximum(mx, jnp.where(m, t1v, NEG))
    mxv = lax.fori_loop(0, nv, _a, jnp.full((L,), NEG, f32))
    mxb[...] = mxv
    pltpu.sync_copy(mxb, sp_m.at[pl.ds(sid * L, L)])
    plsc.subcore_barrier()
    def _r(j, g):
        pltpu.sync_copy(sp_m.at[pl.ds(j * L, L)], mxb)
        return jnp.maximum(g, mxb[...])
    gmax = jnp.max(lax.fori_loop(0, NT, _r, jnp.full((L,), NEG, f32)))

    def _b(i, carry):
        a0, a1, a2, a3, dn = carry
        m = (i * L + i16) < k
        s = jnp.where(m, mylist[pl.ds(i * L, L)], 0)
        pltpu.async_copy(h1_hbm.at[s], xrows, sem).wait()
        t1v = plsc.load_gather(xrows, [i16, col64])
        ex = jnp.where(m, jnp.exp(t1v - gmax), 0.0)
        dn = dn + jnp.sum(ex)
        for r in range(L):
            wr = ex[r]
            a0 = a0 + wr * xrows[r, pl.ds(0, L)]
            a1 = a1 + wr * xrows[r, pl.ds(L, L)]
            a2 = a2 + wr * xrows[r, pl.ds(2 * L, L)]
            a3 = a3 + wr * xrows[r, pl.ds(3 * L, L)]
        return (a0, a1, a2, a3, dn)
    a0, a1, a2, a3, dn = lax.fori_loop(
        0, nv, _b, (zf, zf, zf, zf, jnp.asarray(0.0, f32)))
    hac[pl.ds(0, L)] = a0
    hac[pl.ds(L, L)] = a1
    hac[pl.ds(2 * L, L)] = a2
    hac[pl.ds(3 * L, L)] = a3
    mxb[...] = jnp.full((L,), 0.0, f32) + dn
    pltpu.sync_copy(hac, sp_h.at[pl.ds(sid * 64, 64)])
    pltpu.sync_copy(mxb, sp_d.at[pl.ds(sid * L, L)])
    plsc.subcore_barrier()

    @pl.when(sid == 0)
    def _final():
        def _red(j, carry):
            c0, c1, c2, c3, dt = carry
            pltpu.sync_copy(sp_h.at[pl.ds(j * 64, 64)], hac)
            pltpu.sync_copy(sp_d.at[pl.ds(j * L, L)], mxb)
            return (c0 + hac[pl.ds(0, L)], c1 + hac[pl.ds(L, L)],
                    c2 + hac[pl.ds(2 * L, L)], c3 + hac[pl.ds(3 * L, L)],
                    dt + mxb[pl.ds(0, L)][0])
        c0, c1, c2, c3, dt = lax.fori_loop(
            0, NT, _red, (zf, zf, zf, zf, jnp.asarray(0.0, f32)))
        inv = 1.0 / jnp.maximum(jnp.full((L,), 0.0, f32) + dt, 1e-16)
        pltpu.async_copy(
            h1_hbm.at[jnp.full((L,), DST_NODE, i32)], xrows, sem).wait()
        xcat[pl.ds(0, L)] = xrows[0, pl.ds(0, L)]
        xcat[pl.ds(L, L)] = xrows[0, pl.ds(L, L)]
        xcat[pl.ds(2 * L, L)] = xrows[0, pl.ds(2 * L, L)]
        xcat[pl.ds(3 * L, L)] = xrows[0, pl.ds(3 * L, L)]
        xcat[pl.ds(4 * L, L)] = c0 * inv
        xcat[pl.ds(5 * L, L)] = c1 * inv
        xcat[pl.ds(6 * L, L)] = c2 * inv
        xcat[pl.ds(7 * L, L)] = c3 * inv
        pltpu.sync_copy(w1_hbm, w1v)
        pltpu.sync_copy(b1_hbm, b1v)
        pltpu.sync_copy(wo_hbm, wov)
        pltpu.sync_copy(bo_hbm, bov)
        def _mk(kk, acc):
            d0, d1 = acc
            xk = plsc.load_gather(xcat, [jnp.full((L,), 0, i32) + kk])
            return (d0 + xk * w1v[pl.ds(kk * 32, L)],
                    d1 + xk * w1v[pl.ds(kk * 32 + L, L)])
        d0, d1 = lax.fori_loop(0, 128, _mk, (zf, zf))
        h2b[pl.ds(0, L)] = jnp.maximum(d0 + b1v[pl.ds(0, L)], 0.0)
        h2b[pl.ds(L, L)] = jnp.maximum(d1 + b1v[pl.ds(L, L)], 0.0)
        def _mo(kk, acc):
            hk = plsc.load_gather(h2b, [jnp.full((L,), 0, i32) + kk])
            return tuple(acc[j] + hk * wov[pl.ds(kk * 128 + j * L, L)]
                         for j in range(8))
        o = lax.fori_loop(0, 32, _mo, (zf,) * 8)
        for j in range(8):
            outb[pl.ds(j * L, L)] = o[j] + bov[pl.ds(j * L, L)]
        pltpu.sync_copy(outb, out_hbm)


def _sc_layer2(sel14, counts, h1ext, W1, b1, Wout, bout):
    mesh = plsc.VectorSubcoreMesh(
        core_axis_name="c", subcore_axis_name="s", num_cores=1)
    fn = pl.kernel(
        _sc_layer2_body,
        out_type=jax.ShapeDtypeStruct((128,), f32),
        mesh=mesh,
        scratch_types=[
            pltpu.VMEM((CAP,), i32),
            pltpu.VMEM((L, 128), f32),
            pltpu.VMEM((L,), i32),
            pltpu.VMEM((L,), f32),
            pltpu.VMEM((64,), f32),
            pltpu.VMEM((128 * 32,), f32),
            pltpu.VMEM((32,), f32),
            pltpu.VMEM((32 * 128,), f32),
            pltpu.VMEM((128,), f32),
            pltpu.VMEM((128,), f32),
            pltpu.VMEM((32,), f32),
            pltpu.VMEM((128,), f32),
            pltpu.VMEM_SHARED((NT * L,), f32),
            pltpu.VMEM_SHARED((NT * 64,), f32),
            pltpu.VMEM_SHARED((NT * L,), f32),
            pltpu.SemaphoreType.DMA,
        ],
        compiler_params=pltpu.CompilerParams(needs_layout_passes=False),
    )
    return fn(sel14, counts, h1ext, W1.reshape(-1), b1, Wout.reshape(-1),
              bout)


def kernel(X_v, edge_index, attW0, attq0, W0, b0, attW1, attq1, W1, b1,
           Wout, bout):
    src = edge_index[0].astype(i32)
    dst = edge_index[1].astype(i32)
    xp = jnp.pad(X_v, ((0, NPAD - N), (0, 0)))
    t0 = _tc_scores(xp, attW0, attq0).reshape(NPAD)
    hn, sel14, counts = _sc_layer1(src, dst, t0, X_v)
    h1ext = _tc_hidden(xp, hn, W0, b0, attW1, attq1)
    return _sc_layer2(sel14, counts, h1ext, W1, b1, Wout, bout)
